# baseline (speedup 1.0000x reference)
import functools

import jax
import jax.numpy as jnp
from jax import lax
from jax.experimental import pallas as pl
from jax.experimental.pallas import tpu as pltpu
from jax.experimental.pallas import tpu_sc as plsc

N = 10000
E = 320000
D = 128
H = 20

Hp = 32
NSH = 10240
NC = 2
NS = 16
NW = NC * NS
G = 88
EPAD = NW * G * 128
EW = EPAD // NW
CH = 8
RN = NSH // NS
RB = 1280
NB = NSH // RB

_MESH = dict(core_axis_name="c", subcore_axis_name="s",
             num_cores=NC, num_subcores=NS)


def _deg_pass(dst_all):
    @functools.partial(
        pl.kernel,
        out_type=jax.ShapeDtypeStruct((NW, NSH), jnp.float32),
        mesh=plsc.VectorSubcoreMesh(**_MESH),
        scratch_types=[
            pltpu.VMEM((EW,), jnp.int32),
            pltpu.VMEM((NSH,), jnp.float32),
        ],
        compiler_params=pltpu.CompilerParams(needs_layout_passes=False),
    )
    def k(dst_hbm, out_hbm, idx_v, deg_v):
        c = lax.axis_index("c")
        s = lax.axis_index("s")
        wid = s * NC + c
        z = jnp.zeros((16,), jnp.float32)

        def zero(i, carry):
            deg_v[pl.ds(i * 16, 16)] = z
            return carry

        lax.fori_loop(0, NSH // 16, zero, 0)
        pltpu.sync_copy(dst_hbm.at[pl.ds(wid * EW, EW)], idx_v)
        ones = jnp.ones((16,), jnp.float32)

        def body(i, carry):
            idx = idx_v[pl.ds(i * 16, 16)]
            plsc.addupdate_scatter(deg_v, [idx], ones)
            return carry

        lax.fori_loop(0, EW // 16, body, 0)
        pltpu.sync_copy(deg_v, out_hbm.at[wid])

    return k(dst_all)


def _agg_pass(g, src2d, dst2d):
    @functools.partial(
        pl.kernel,
        out_type=jax.ShapeDtypeStruct((NC, NSH, 128), jnp.float32),
        mesh=plsc.VectorSubcoreMesh(**_MESH),
        scratch_types=[
            pltpu.VMEM((CH, 128), jnp.int32),
            pltpu.VMEM((CH, 128), jnp.int32),
            pltpu.VMEM((128, 128), jnp.float32),
            pltpu.VMEM((128, 128), jnp.float32),
            pltpu.SemaphoreType.DMA,
            pltpu.SemaphoreType.DMA,
            pltpu.SemaphoreType.DMA,
            pltpu.SemaphoreType.DMA,
            pltpu.VMEM_SHARED((NSH, 128), jnp.float32),
        ],
    )
    def k(g_hbm, src_hbm, dst_hbm, out_hbm, src_v, dst_v, rows0_v, rows1_v,
          sg0, sg1, ss0, ss1, acc_sh):
        c = lax.axis_index("c")
        s = lax.axis_index("s")
        wid = s * NC + c
        z = jnp.zeros((16,), jnp.float32)

        def zrow(i, carry):
            for t in range(8):
                rows0_v[i, pl.ds(t * 16, 16)] = z
            return carry

        lax.fori_loop(0, 128, zrow, 0)

        def zcopy(i, carry):
            pltpu.sync_copy(rows0_v, acc_sh.at[pl.ds(s * RN + i * 128, 128)])
            return carry

        lax.fori_loop(0, RN // 128, zcopy, 0)
        plsc.subcore_barrier()

        def chunk(ci, carry):
            base = wid * G + ci * CH
            pltpu.sync_copy(src_hbm.at[pl.ds(base, CH)], src_v)
            pltpu.sync_copy(dst_hbm.at[pl.ds(base, CH)], dst_v)
            pltpu.async_copy(g_hbm.at[src_v.at[0]], rows0_v, sg0)

            def pair(p, carry2):
                j0 = 2 * p
                j1 = j0 + 1
                cg1 = pltpu.async_copy(g_hbm.at[src_v.at[j1]], rows1_v, sg1)
                pltpu.make_async_copy(
                    g_hbm.at[src_v.at[j0]], rows0_v, sg0).wait()
                pltpu.async_copy(
                    rows0_v, acc_sh.at[dst_v.at[j0]], ss0, add=True).wait()

                @pl.when(p < CH // 2 - 1)
                def _():
                    pltpu.async_copy(g_hbm.at[src_v.at[j0 + 2]], rows0_v, sg0)

                cg1.wait()
                pltpu.async_copy(
                    rows1_v, acc_sh.at[dst_v.at[j1]], ss1, add=True).wait()
                return carry2

            lax.fori_loop(0, CH // 2, pair, 0)
            return carry

        lax.fori_loop(0, G // CH, chunk, 0)
        plsc.subcore_barrier()

        def wback(i, carry):
            pltpu.sync_copy(acc_sh.at[pl.ds(s * RN + i * 128, 128)], rows0_v)
            pltpu.sync_copy(rows0_v,
                            out_hbm.at[c, pl.ds(s * RN + i * 128, 128)])
            return carry

        lax.fori_loop(0, RN // 128, wback, 0)

    return k(g, src2d, dst2d)


def _elu(v):
    return jnp.where(v > 0, v, jnp.exp(v) - 1.0)


def _dense1(xp, W1p, deg_parts):
    def body(x_ref, w_ref, dp_ref, g_ref, dinv_ref):
        h = lax.dot_general(x_ref[...], w_ref[...], (((1,), (1,)), ((), ())),
                            preferred_element_type=jnp.float32)
        deg = jnp.sum(dp_ref[...], axis=0)
        dinv = lax.rsqrt(deg)
        g_ref[...] = jnp.concatenate(
            [h * dinv[:, None], jnp.zeros((RB, 128 - Hp), jnp.float32)],
            axis=1)
        dinv_ref[...] = dinv[:, None]

    return pl.pallas_call(
        body,
        grid=(NB,),
        in_specs=[
            pl.BlockSpec((RB, D), lambda i: (i, 0)),
            pl.BlockSpec((Hp, D), lambda i: (0, 0)),
            pl.BlockSpec((NW, RB), lambda i: (0, i)),
        ],
        out_specs=[
            pl.BlockSpec((RB, 128), lambda i: (i, 0)),
            pl.BlockSpec((RB, 1), lambda i: (i, 0)),
        ],
        out_shape=[
            jax.ShapeDtypeStruct((NSH, 128), jnp.float32),
            jax.ShapeDtypeStruct((NSH, 1), jnp.float32),
        ],
    )(xp, W1p, deg_parts)


def _dense2(a0, a1, dinv, b1p, W2p):
    def body(a0_ref, a1_ref, dv_ref, b_ref, w_ref, g_ref):
        dv = dv_ref[...]
        a = a0_ref[...] + a1_ref[...]
        h2 = _elu(dv * a[:, :Hp] + b_ref[...])
        t = lax.dot_general(h2, w_ref[...], (((1,), (1,)), ((), ())),
                            preferred_element_type=jnp.float32)
        g_ref[...] = jnp.concatenate(
            [t * dv, jnp.zeros((RB, 128 - Hp), jnp.float32)], axis=1)

    return pl.pallas_call(
        body,
        grid=(NB,),
        in_specs=[
            pl.BlockSpec((RB, 128), lambda i: (i, 0)),
            pl.BlockSpec((RB, 128), lambda i: (i, 0)),
            pl.BlockSpec((RB, 1), lambda i: (i, 0)),
            pl.BlockSpec((1, Hp), lambda i: (0, 0)),
            pl.BlockSpec((Hp, Hp), lambda i: (0, 0)),
        ],
        out_specs=pl.BlockSpec((RB, 128), lambda i: (i, 0)),
        out_shape=jax.ShapeDtypeStruct((NSH, 128), jnp.float32),
    )(a0, a1, dinv, b1p, W2p)


def _dense3(a0, a1, dinv, b2p, Wd1p, bd1p, Wd2p, bd2b):
    def body(a0_ref, a1_ref, dv_ref, b2_ref, w1_ref, c1_ref, w2_ref, c2_ref,
             o_ref):
        dv = dv_ref[...]
        a = a0_ref[...] + a1_ref[...]
        h3 = _elu(dv * a[:, :Hp] + b2_ref[...])
        d1 = lax.dot_general(h3, w1_ref[...], (((1,), (1,)), ((), ())),
                             preferred_element_type=jnp.float32)
        d1 = jnp.maximum(d1 + c1_ref[...], 0.0)
        o = lax.dot_general(d1, w2_ref[...], (((1,), (1,)), ((), ())),
                            preferred_element_type=jnp.float32)
        o_ref[...] = o + c2_ref[...]

    return pl.pallas_call(
        body,
        grid=(NB,),
        in_specs=[
            pl.BlockSpec((RB, 128), lambda i: (i, 0)),
            pl.BlockSpec((RB, 128), lambda i: (i, 0)),
            pl.BlockSpec((RB, 1), lambda i: (i, 0)),
            pl.BlockSpec((1, Hp), lambda i: (0, 0)),
            pl.BlockSpec((Hp, Hp), lambda i: (0, 0)),
            pl.BlockSpec((1, Hp), lambda i: (0, 0)),
            pl.BlockSpec((8, Hp), lambda i: (0, 0)),
            pl.BlockSpec((1, 8), lambda i: (0, 0)),
        ],
        out_specs=pl.BlockSpec((RB, 8), lambda i: (i, 0)),
        out_shape=jax.ShapeDtypeStruct((NSH, 8), jnp.float32),
    )(a0, a1, dinv, b2p, Wd1p, bd1p, Wd2p, bd2b)


def kernel(x, edge_index, W1, b1, W2, b2, Wd1, bd1, Wd2, bd2):
    loop = jnp.arange(N, dtype=jnp.int32)
    padi = jnp.full((EPAD - E - N,), N, jnp.int32)
    src_all = jnp.concatenate([edge_index[0], loop, padi])
    dst_all = jnp.concatenate([edge_index[1], loop, padi])
    src2d = src_all.reshape(EPAD // 128, 128)
    dst2d = dst_all.reshape(EPAD // 128, 128)

    xp = jnp.pad(x, ((0, NSH - N), (0, 0)))
    W1p = jnp.pad(W1, ((0, Hp - H), (0, 0)))
    W2p = jnp.pad(W2, ((0, Hp - H), (0, Hp - H)))
    b1p = jnp.pad(b1, (0, Hp - H)).reshape(1, Hp)
    b2p = jnp.pad(b2, (0, Hp - H)).reshape(1, Hp)
    Wd1p = jnp.pad(Wd1, ((0, Hp - 10), (0, Hp - H)))
    bd1p = jnp.pad(bd1, (0, Hp - 10)).reshape(1, Hp)
    Wd2p = jnp.pad(Wd2, ((0, 8 - 1), (0, Hp - 10)))
    bd2b = jnp.broadcast_to(bd2, (8,)).reshape(1, 8)

    deg_parts = _deg_pass(dst_all)
    g1, dinv = _dense1(xp, W1p, deg_parts)
    agg1 = _agg_pass(g1, src2d, dst2d)
    g2 = _dense2(agg1[0], agg1[1], dinv, b1p, W2p)
    agg2 = _agg_pass(g2, src2d, dst2d)
    out8 = _dense3(agg2[0], agg2[1], dinv, b2p, Wd1p, bd1p, Wd2p, bd2b)
    return out8[:N, :1]


# speedup vs baseline: 3.5109x; 3.5109x over previous
import functools

import jax
import jax.numpy as jnp
from jax import lax
from jax.experimental import pallas as pl
from jax.experimental.pallas import tpu as pltpu
from jax.experimental.pallas import tpu_sc as plsc

N = 10000
E = 320000
D = 128
H = 20

Hp = 32
NSH = 10240
NC = 2
NS = 16
NW = NC * NS
G = 88
EPAD = NW * G * 128
EW = EPAD // NW
CH = 8
RN = NSH // NS
RB = 1280
NB = NSH // RB

_MESH = dict(core_axis_name="c", subcore_axis_name="s",
             num_cores=NC, num_subcores=NS)


def _deg_pass(dst_all):
    @functools.partial(
        pl.kernel,
        out_type=jax.ShapeDtypeStruct((NW, NSH), jnp.float32),
        mesh=plsc.VectorSubcoreMesh(**_MESH),
        scratch_types=[
            pltpu.VMEM((EW,), jnp.int32),
            pltpu.VMEM((NSH,), jnp.float32),
        ],
        compiler_params=pltpu.CompilerParams(needs_layout_passes=False),
    )
    def k(dst_hbm, out_hbm, idx_v, deg_v):
        c = lax.axis_index("c")
        s = lax.axis_index("s")
        wid = s * NC + c
        z = jnp.zeros((16,), jnp.float32)

        def zero(i, carry):
            deg_v[pl.ds(i * 16, 16)] = z
            return carry

        lax.fori_loop(0, NSH // 16, zero, 0)
        pltpu.sync_copy(dst_hbm.at[pl.ds(wid * EW, EW)], idx_v)
        ones = jnp.ones((16,), jnp.float32)

        def body(i, carry):
            idx = idx_v[pl.ds(i * 16, 16)]
            plsc.addupdate_scatter(deg_v, [idx], ones)
            return carry

        lax.fori_loop(0, EW // 16, body, 0)
        pltpu.sync_copy(deg_v, out_hbm.at[wid])

    return k(dst_all)


def _agg_pass(g, src2d, dst2d):
    @functools.partial(
        pl.kernel,
        out_type=jax.ShapeDtypeStruct((NC, NSH, Hp), jnp.float32),
        mesh=plsc.VectorSubcoreMesh(**_MESH),
        compiler_params=pltpu.CompilerParams(use_tc_tiling_on_sc=False),
        scratch_types=[
            pltpu.VMEM((CH, 128), jnp.int32),
            pltpu.VMEM((CH, 128), jnp.int32),
            pltpu.VMEM((128, Hp), jnp.float32),
            pltpu.VMEM((128, Hp), jnp.float32),
            pltpu.SemaphoreType.DMA,
            pltpu.SemaphoreType.DMA,
            pltpu.SemaphoreType.DMA,
            pltpu.SemaphoreType.DMA,
            pltpu.VMEM_SHARED((NSH, Hp), jnp.float32),
        ],
    )
    def k(g_hbm, src_hbm, dst_hbm, out_hbm, src_v, dst_v, rows0_v, rows1_v,
          sg0, sg1, ss0, ss1, acc_sh):
        c = lax.axis_index("c")
        s = lax.axis_index("s")
        wid = s * NC + c
        z = jnp.zeros((16,), jnp.float32)

        def zrow(i, carry):
            for t in range(Hp // 16):
                rows0_v[i, pl.ds(t * 16, 16)] = z
            return carry

        lax.fori_loop(0, 128, zrow, 0)

        def zcopy(i, carry):
            pltpu.sync_copy(rows0_v, acc_sh.at[pl.ds(s * RN + i * 128, 128)])
            return carry

        lax.fori_loop(0, RN // 128, zcopy, 0)
        plsc.subcore_barrier()

        def chunk(ci, carry):
            base = wid * G + ci * CH
            pltpu.sync_copy(src_hbm.at[pl.ds(base, CH)], src_v)
            pltpu.sync_copy(dst_hbm.at[pl.ds(base, CH)], dst_v)
            pltpu.async_copy(g_hbm.at[src_v.at[0]], rows0_v, sg0)

            def pair(p, carry2):
                j0 = 2 * p
                j1 = j0 + 1
                cg1 = pltpu.async_copy(g_hbm.at[src_v.at[j1]], rows1_v, sg1)
                pltpu.make_async_copy(
                    g_hbm.at[src_v.at[j0]], rows0_v, sg0).wait()
                pltpu.async_copy(
                    rows0_v, acc_sh.at[dst_v.at[j0]], ss0, add=True).wait()

                @pl.when(p < CH // 2 - 1)
                def _():
                    pltpu.async_copy(g_hbm.at[src_v.at[j0 + 2]], rows0_v, sg0)

                cg1.wait()
                pltpu.async_copy(
                    rows1_v, acc_sh.at[dst_v.at[j1]], ss1, add=True).wait()
                return carry2

            lax.fori_loop(0, CH // 2, pair, 0)
            return carry

        lax.fori_loop(0, G // CH, chunk, 0)
        plsc.subcore_barrier()

        def wback(i, carry):
            pltpu.sync_copy(acc_sh.at[pl.ds(s * RN + i * 128, 128)], rows0_v)
            pltpu.sync_copy(rows0_v,
                            out_hbm.at[c, pl.ds(s * RN + i * 128, 128)])
            return carry

        lax.fori_loop(0, RN // 128, wback, 0)

    return k(g, src2d, dst2d)


def _elu(v):
    return jnp.where(v > 0, v, jnp.exp(v) - 1.0)


def _dense1(xp, W1p, deg_parts):
    def body(x_ref, w_ref, dp_ref, g_ref, dinv_ref):
        h = lax.dot_general(x_ref[...], w_ref[...], (((1,), (1,)), ((), ())),
                            preferred_element_type=jnp.float32)
        deg = jnp.sum(dp_ref[...], axis=0)
        dinv = lax.rsqrt(deg)
        g_ref[...] = h * dinv[:, None]
        dinv_ref[...] = dinv[:, None]

    return pl.pallas_call(
        body,
        grid=(NB,),
        in_specs=[
            pl.BlockSpec((RB, D), lambda i: (i, 0)),
            pl.BlockSpec((Hp, D), lambda i: (0, 0)),
            pl.BlockSpec((NW, RB), lambda i: (0, i)),
        ],
        out_specs=[
            pl.BlockSpec((RB, Hp), lambda i: (i, 0)),
            pl.BlockSpec((RB, 1), lambda i: (i, 0)),
        ],
        out_shape=[
            jax.ShapeDtypeStruct((NSH, Hp), jnp.float32),
            jax.ShapeDtypeStruct((NSH, 1), jnp.float32),
        ],
    )(xp, W1p, deg_parts)


def _dense2(a0, a1, dinv, b1p, W2p):
    def body(a0_ref, a1_ref, dv_ref, b_ref, w_ref, g_ref):
        dv = dv_ref[...]
        h2 = _elu(dv * (a0_ref[...] + a1_ref[...]) + b_ref[...])
        t = lax.dot_general(h2, w_ref[...], (((1,), (1,)), ((), ())),
                            preferred_element_type=jnp.float32)
        g_ref[...] = t * dv

    return pl.pallas_call(
        body,
        grid=(NB,),
        in_specs=[
            pl.BlockSpec((RB, Hp), lambda i: (i, 0)),
            pl.BlockSpec((RB, Hp), lambda i: (i, 0)),
            pl.BlockSpec((RB, 1), lambda i: (i, 0)),
            pl.BlockSpec((1, Hp), lambda i: (0, 0)),
            pl.BlockSpec((Hp, Hp), lambda i: (0, 0)),
        ],
        out_specs=pl.BlockSpec((RB, Hp), lambda i: (i, 0)),
        out_shape=jax.ShapeDtypeStruct((NSH, Hp), jnp.float32),
    )(a0, a1, dinv, b1p, W2p)


def _dense3(a0, a1, dinv, b2p, Wd1p, bd1p, Wd2p, bd2b):
    def body(a0_ref, a1_ref, dv_ref, b2_ref, w1_ref, c1_ref, w2_ref, c2_ref,
             o_ref):
        dv = dv_ref[...]
        h3 = _elu(dv * (a0_ref[...] + a1_ref[...]) + b2_ref[...])
        d1 = lax.dot_general(h3, w1_ref[...], (((1,), (1,)), ((), ())),
                             preferred_element_type=jnp.float32)
        d1 = jnp.maximum(d1 + c1_ref[...], 0.0)
        o = lax.dot_general(d1, w2_ref[...], (((1,), (1,)), ((), ())),
                            preferred_element_type=jnp.float32)
        o_ref[...] = o + c2_ref[...]

    return pl.pallas_call(
        body,
        grid=(NB,),
        in_specs=[
            pl.BlockSpec((RB, Hp), lambda i: (i, 0)),
            pl.BlockSpec((RB, Hp), lambda i: (i, 0)),
            pl.BlockSpec((RB, 1), lambda i: (i, 0)),
            pl.BlockSpec((1, Hp), lambda i: (0, 0)),
            pl.BlockSpec((Hp, Hp), lambda i: (0, 0)),
            pl.BlockSpec((1, Hp), lambda i: (0, 0)),
            pl.BlockSpec((8, Hp), lambda i: (0, 0)),
            pl.BlockSpec((1, 8), lambda i: (0, 0)),
        ],
        out_specs=pl.BlockSpec((RB, 8), lambda i: (i, 0)),
        out_shape=jax.ShapeDtypeStruct((NSH, 8), jnp.float32),
    )(a0, a1, dinv, b2p, Wd1p, bd1p, Wd2p, bd2b)


def kernel(x, edge_index, W1, b1, W2, b2, Wd1, bd1, Wd2, bd2):
    loop = jnp.arange(N, dtype=jnp.int32)
    padi = jnp.full((EPAD - E - N,), N, jnp.int32)
    src_all = jnp.concatenate([edge_index[0], loop, padi])
    dst_all = jnp.concatenate([edge_index[1], loop, padi])
    src2d = src_all.reshape(EPAD // 128, 128)
    dst2d = dst_all.reshape(EPAD // 128, 128)

    xp = jnp.pad(x, ((0, NSH - N), (0, 0)))
    W1p = jnp.pad(W1, ((0, Hp - H), (0, 0)))
    W2p = jnp.pad(W2, ((0, Hp - H), (0, Hp - H)))
    b1p = jnp.pad(b1, (0, Hp - H)).reshape(1, Hp)
    b2p = jnp.pad(b2, (0, Hp - H)).reshape(1, Hp)
    Wd1p = jnp.pad(Wd1, ((0, Hp - 10), (0, Hp - H)))
    bd1p = jnp.pad(bd1, (0, Hp - 10)).reshape(1, Hp)
    Wd2p = jnp.pad(Wd2, ((0, 8 - 1), (0, Hp - 10)))
    bd2b = jnp.broadcast_to(bd2, (8,)).reshape(1, 8)

    deg_parts = _deg_pass(dst_all)
    g1, dinv = _dense1(xp, W1p, deg_parts)
    agg1 = _agg_pass(g1, src2d, dst2d)
    g2 = _dense2(agg1[0], agg1[1], dinv, b1p, W2p)
    agg2 = _agg_pass(g2, src2d, dst2d)
    out8 = _dense3(agg2[0], agg2[1], dinv, b2p, Wd1p, bd1p, Wd2p, bd2b)
    return out8[:N, :1]


# speedup vs baseline: 3.5374x; 1.0075x over previous
import functools

import jax
import jax.numpy as jnp
from jax import lax
from jax.experimental import pallas as pl
from jax.experimental.pallas import tpu as pltpu
from jax.experimental.pallas import tpu_sc as plsc

N = 10000
E = 320000
D = 128
H = 20

Hp = 32
NSH = 10240
NC = 2
NS = 16
NW = NC * NS
G = 88
EPAD = NW * G * 128
EW = EPAD // NW
CH = 8
RN = NSH // NS
RB = 1280
NB = NSH // RB

_MESH = dict(core_axis_name="c", subcore_axis_name="s",
             num_cores=NC, num_subcores=NS)


def _deg_pass(dst_all):
    @functools.partial(
        pl.kernel,
        out_type=jax.ShapeDtypeStruct((NW, NSH), jnp.float32),
        mesh=plsc.VectorSubcoreMesh(**_MESH),
        scratch_types=[
            pltpu.VMEM((EW,), jnp.int32),
            pltpu.VMEM((NSH,), jnp.float32),
        ],
        compiler_params=pltpu.CompilerParams(needs_layout_passes=False),
    )
    def k(dst_hbm, out_hbm, idx_v, deg_v):
        c = lax.axis_index("c")
        s = lax.axis_index("s")
        wid = s * NC + c
        z = jnp.zeros((16,), jnp.float32)

        def zero(i, carry):
            deg_v[pl.ds(i * 16, 16)] = z
            return carry

        lax.fori_loop(0, NSH // 16, zero, 0)
        pltpu.sync_copy(dst_hbm.at[pl.ds(wid * EW, EW)], idx_v)
        ones = jnp.ones((16,), jnp.float32)

        def body(i, carry):
            idx = idx_v[pl.ds(i * 16, 16)]
            plsc.addupdate_scatter(deg_v, [idx], ones)
            return carry

        lax.fori_loop(0, EW // 16, body, 0)
        pltpu.sync_copy(deg_v, out_hbm.at[wid])

    return k(dst_all)


def _agg_pass(g, src2d, dst2d):
    @functools.partial(
        pl.kernel,
        out_type=jax.ShapeDtypeStruct((NC, NSH, Hp), jnp.float32),
        mesh=plsc.VectorSubcoreMesh(**_MESH),
        compiler_params=pltpu.CompilerParams(use_tc_tiling_on_sc=False),
        scratch_types=[
            pltpu.VMEM((G, 128), jnp.int32),
            pltpu.VMEM((G, 128), jnp.int32),
            [pltpu.VMEM((128, Hp), jnp.float32)] * 4,
            [pltpu.SemaphoreType.DMA] * 4,
            [pltpu.SemaphoreType.DMA] * 4,
            pltpu.VMEM_SHARED((NSH, Hp), jnp.float32),
        ],
    )
    def k(g_hbm, src_hbm, dst_hbm, out_hbm, src_v, dst_v, rows, sg, ss,
          acc_sh):
        c = lax.axis_index("c")
        s = lax.axis_index("s")
        wid = s * NC + c
        pltpu.sync_copy(src_hbm.at[pl.ds(wid * G, G)], src_v)
        pltpu.sync_copy(dst_hbm.at[pl.ds(wid * G, G)], dst_v)
        z = jnp.zeros((16,), jnp.float32)

        def zrow(i, carry):
            for t in range(Hp // 16):
                rows[0][i, pl.ds(t * 16, 16)] = z
            return carry

        lax.fori_loop(0, 128, zrow, 0)

        def zcopy(i, carry):
            pltpu.sync_copy(rows[0], acc_sh.at[pl.ds(s * RN + i * 128, 128)])
            return carry

        lax.fori_loop(0, RN // 128, zcopy, 0)
        plsc.subcore_barrier()

        def quad(q, carry):
            for t in range(4):
                @pl.when(q > 0)
                def _():
                    pltpu.make_async_copy(
                        rows[t], acc_sh.at[dst_v.at[4 * (q - 1) + t]],
                        ss[t]).wait()
                pltpu.async_copy(g_hbm.at[src_v.at[4 * q + t]], rows[t],
                                 sg[t])
            for t in range(4):
                pltpu.make_async_copy(g_hbm.at[src_v.at[4 * q + t]], rows[t],
                                      sg[t]).wait()
                pltpu.async_copy(rows[t], acc_sh.at[dst_v.at[4 * q + t]],
                                 ss[t], add=True)
            return carry

        lax.fori_loop(0, G // 4, quad, 0)
        for t in range(4):
            pltpu.make_async_copy(
                rows[t], acc_sh.at[dst_v.at[G - 4 + t]], ss[t]).wait()
        plsc.subcore_barrier()

        def wback(i, carry):
            pltpu.sync_copy(acc_sh.at[pl.ds(s * RN + i * 128, 128)], rows[0])
            pltpu.sync_copy(rows[0],
                            out_hbm.at[c, pl.ds(s * RN + i * 128, 128)])
            return carry

        lax.fori_loop(0, RN // 128, wback, 0)

    return k(g, src2d, dst2d)


def _elu(v):
    return jnp.where(v > 0, v, jnp.exp(v) - 1.0)


def _dense1(xp, W1p, deg_parts):
    def body(x_ref, w_ref, dp_ref, g_ref, dinv_ref):
        h = lax.dot_general(x_ref[...], w_ref[...], (((1,), (1,)), ((), ())),
                            preferred_element_type=jnp.float32)
        deg = jnp.sum(dp_ref[...], axis=0)
        dinv = lax.rsqrt(deg)
        g_ref[...] = h * dinv[:, None]
        dinv_ref[...] = dinv[:, None]

    return pl.pallas_call(
        body,
        grid=(NB,),
        in_specs=[
            pl.BlockSpec((RB, D), lambda i: (i, 0)),
            pl.BlockSpec((Hp, D), lambda i: (0, 0)),
            pl.BlockSpec((NW, RB), lambda i: (0, i)),
        ],
        out_specs=[
            pl.BlockSpec((RB, Hp), lambda i: (i, 0)),
            pl.BlockSpec((RB, 1), lambda i: (i, 0)),
        ],
        out_shape=[
            jax.ShapeDtypeStruct((NSH, Hp), jnp.float32),
            jax.ShapeDtypeStruct((NSH, 1), jnp.float32),
        ],
    )(xp, W1p, deg_parts)


def _dense2(a0, a1, dinv, b1p, W2p):
    def body(a0_ref, a1_ref, dv_ref, b_ref, w_ref, g_ref):
        dv = dv_ref[...]
        h2 = _elu(dv * (a0_ref[...] + a1_ref[...]) + b_ref[...])
        t = lax.dot_general(h2, w_ref[...], (((1,), (1,)), ((), ())),
                            preferred_element_type=jnp.float32)
        g_ref[...] = t * dv

    return pl.pallas_call(
        body,
        grid=(NB,),
        in_specs=[
            pl.BlockSpec((RB, Hp), lambda i: (i, 0)),
            pl.BlockSpec((RB, Hp), lambda i: (i, 0)),
            pl.BlockSpec((RB, 1), lambda i: (i, 0)),
            pl.BlockSpec((1, Hp), lambda i: (0, 0)),
            pl.BlockSpec((Hp, Hp), lambda i: (0, 0)),
        ],
        out_specs=pl.BlockSpec((RB, Hp), lambda i: (i, 0)),
        out_shape=jax.ShapeDtypeStruct((NSH, Hp), jnp.float32),
    )(a0, a1, dinv, b1p, W2p)


def _dense3(a0, a1, dinv, b2p, Wd1p, bd1p, Wd2p, bd2b):
    def body(a0_ref, a1_ref, dv_ref, b2_ref, w1_ref, c1_ref, w2_ref, c2_ref,
             o_ref):
        dv = dv_ref[...]
        h3 = _elu(dv * (a0_ref[...] + a1_ref[...]) + b2_ref[...])
        d1 = lax.dot_general(h3, w1_ref[...], (((1,), (1,)), ((), ())),
                             preferred_element_type=jnp.float32)
        d1 = jnp.maximum(d1 + c1_ref[...], 0.0)
        o = lax.dot_general(d1, w2_ref[...], (((1,), (1,)), ((), ())),
                            preferred_element_type=jnp.float32)
        o_ref[...] = o + c2_ref[...]

    return pl.pallas_call(
        body,
        grid=(NB,),
        in_specs=[
            pl.BlockSpec((RB, Hp), lambda i: (i, 0)),
            pl.BlockSpec((RB, Hp), lambda i: (i, 0)),
            pl.BlockSpec((RB, 1), lambda i: (i, 0)),
            pl.BlockSpec((1, Hp), lambda i: (0, 0)),
            pl.BlockSpec((Hp, Hp), lambda i: (0, 0)),
            pl.BlockSpec((1, Hp), lambda i: (0, 0)),
            pl.BlockSpec((8, Hp), lambda i: (0, 0)),
            pl.BlockSpec((1, 8), lambda i: (0, 0)),
        ],
        out_specs=pl.BlockSpec((RB, 8), lambda i: (i, 0)),
        out_shape=jax.ShapeDtypeStruct((NSH, 8), jnp.float32),
    )(a0, a1, dinv, b2p, Wd1p, bd1p, Wd2p, bd2b)


def kernel(x, edge_index, W1, b1, W2, b2, Wd1, bd1, Wd2, bd2):
    loop = jnp.arange(N, dtype=jnp.int32)
    padi = jnp.full((EPAD - E - N,), N, jnp.int32)
    src_all = jnp.concatenate([edge_index[0], loop, padi])
    dst_all = jnp.concatenate([edge_index[1], loop, padi])
    src2d = src_all.reshape(EPAD // 128, 128)
    dst2d = dst_all.reshape(EPAD // 128, 128)

    xp = jnp.pad(x, ((0, NSH - N), (0, 0)))
    W1p = jnp.pad(W1, ((0, Hp - H), (0, 0)))
    W2p = jnp.pad(W2, ((0, Hp - H), (0, Hp - H)))
    b1p = jnp.pad(b1, (0, Hp - H)).reshape(1, Hp)
    b2p = jnp.pad(b2, (0, Hp - H)).reshape(1, Hp)
    Wd1p = jnp.pad(Wd1, ((0, Hp - 10), (0, Hp - H)))
    bd1p = jnp.pad(bd1, (0, Hp - 10)).reshape(1, Hp)
    Wd2p = jnp.pad(Wd2, ((0, 8 - 1), (0, Hp - 10)))
    bd2b = jnp.broadcast_to(bd2, (8,)).reshape(1, 8)

    deg_parts = _deg_pass(dst_all)
    g1, dinv = _dense1(xp, W1p, deg_parts)
    agg1 = _agg_pass(g1, src2d, dst2d)
    g2 = _dense2(agg1[0], agg1[1], dinv, b1p, W2p)
    agg2 = _agg_pass(g2, src2d, dst2d)
    out8 = _dense3(agg2[0], agg2[1], dinv, b2p, Wd1p, bd1p, Wd2p, bd2b)
    return out8[:N, :1]
